# Initial kernel scaffold; baseline (speedup 1.0000x reference)
import jax
import jax.numpy as jnp
from jax.experimental import pallas as pl


def kernel(x, edge_index, edge_attr, W, W_edge, att_src, att_dst, att_edge, bias):
    raise NotImplementedError("write your pallas kernel here")


# speedup vs baseline: 16.5824x; 16.5824x over previous
import functools

import jax
import jax.numpy as jnp
from jax import lax
from jax.experimental import pallas as pl
from jax.experimental.pallas import tpu as pltpu
from jax.experimental.pallas import tpu_sc as plsc

N = 10000
E = 320000
D_IN = 128
D_EDGE = 16
C = 128
NEG_SLOPE = 0.2

NP = 10240
NTILES = 32
EPT = E // NTILES
VPT = EPT // 16
KB = 125
NB = EPT // KB
NBT = E // KB
NODE_CHUNK = NP // 16
DCH = 640
ZROWS = 80

_EBLK = 2560
_GRID_E = E // _EBLK


def _proj_body(x_ref, w_ref, asv_ref, adv_ref, h_ref, as_ref, ad_ref):
    h = lax.dot_general(x_ref[...], w_ref[...], (((1,), (1,)), ((), ())),
                        preferred_element_type=jnp.float32)
    h_ref[...] = h
    as_ref[...] = lax.dot_general(asv_ref[...], h, (((1,), (1,)), ((), ())),
                                  preferred_element_type=jnp.float32)
    ad_ref[...] = lax.dot_general(adv_ref[...], h, (((1,), (1,)), ((), ())),
                                  preferred_element_type=jnp.float32)


_proj = pl.pallas_call(
    _proj_body,
    out_shape=[
        jax.ShapeDtypeStruct((N, C), jnp.float32),
        jax.ShapeDtypeStruct((1, N), jnp.float32),
        jax.ShapeDtypeStruct((1, N), jnp.float32),
    ],
)


def _aedge_body(ea_ref, we_ref, aev_ref, out_ref):
    v = lax.dot_general(aev_ref[...], we_ref[...], (((1,), (0,)), ((), ())),
                        preferred_element_type=jnp.float32)
    ae = lax.dot_general(v, ea_ref[...], (((1,), (1,)), ((), ())),
                         preferred_element_type=jnp.float32)
    out_ref[...] = ae.reshape(1, 1, _EBLK)


_aedge = pl.pallas_call(
    _aedge_body,
    grid=(_GRID_E,),
    in_specs=[
        pl.BlockSpec((_EBLK, D_EDGE), lambda i: (i, 0)),
        pl.BlockSpec((C, D_EDGE), lambda i: (0, 0)),
        pl.BlockSpec((1, C), lambda i: (0, 0)),
    ],
    out_specs=pl.BlockSpec((1, 1, _EBLK), lambda i: (i, 0, 0)),
    out_shape=jax.ShapeDtypeStruct((_GRID_E, 1, _EBLK), jnp.float32),
)


_sc_mesh = plsc.VectorSubcoreMesh(core_axis_name="c", subcore_axis_name="s")


@functools.partial(
    pl.kernel,
    mesh=_sc_mesh,
    out_type=[
        jax.ShapeDtypeStruct((E,), jnp.float32),
        jax.ShapeDtypeStruct((2 * NP,), jnp.float32),
    ],
    compiler_params=pltpu.CompilerParams(needs_layout_passes=False),
    scratch_types=[
        pltpu.VMEM((EPT,), jnp.int32),
        pltpu.VMEM((EPT,), jnp.int32),
        pltpu.VMEM((EPT,), jnp.float32),
        pltpu.VMEM((EPT,), jnp.float32),
        pltpu.VMEM((N,), jnp.float32),
        pltpu.VMEM((N,), jnp.float32),
        pltpu.VMEM((NP,), jnp.float32),
        pltpu.VMEM((16, NODE_CHUNK), jnp.float32),
        pltpu.VMEM((NODE_CHUNK,), jnp.float32),
        pltpu.VMEM_SHARED((16, NP), jnp.float32),
    ],
)
def _attn(asrc_hbm, adst_hbm, ae_hbm, src_hbm, dst_hbm,
          ex_hbm, den_hbm,
          src_v, dst_v, ae_v, ex_v, asrc_v, adst_v, den_v,
          colbuf, red_v, den_sh):
    c = lax.axis_index("c")
    s = lax.axis_index("s")
    wid = s * 2 + c
    base = wid * EPT
    pltpu.sync_copy(src_hbm.at[pl.ds(base, EPT)], src_v)
    pltpu.sync_copy(dst_hbm.at[pl.ds(base, EPT)], dst_v)
    pltpu.sync_copy(ae_hbm.at[pl.ds(base, EPT)], ae_v)
    pltpu.sync_copy(asrc_hbm, asrc_v)
    pltpu.sync_copy(adst_hbm, adst_v)

    zeros = jnp.zeros((16,), jnp.float32)

    def zbody(i, _):
        den_v[pl.ds(i * 16, 16)] = zeros
        return 0

    lax.fori_loop(0, NP // 16, zbody, 0)

    def body(i, _):
        sl = pl.ds(i * 16, 16)
        sv = src_v[sl]
        dv = dst_v[sl]
        a = plsc.load_gather(asrc_v, [sv]) + plsc.load_gather(adst_v, [dv]) + ae_v[sl]
        a = jnp.where(a >= 0, a, NEG_SLOPE * a)
        ev = jnp.exp(a)
        ex_v[sl] = ev
        plsc.addupdate_scatter(den_v, [dv], ev)
        return 0

    lax.fori_loop(0, VPT, body, 0)
    pltpu.sync_copy(ex_v, ex_hbm.at[pl.ds(base, EPT)])

    pltpu.sync_copy(den_v, den_sh.at[s])
    plsc.subcore_barrier()
    pltpu.sync_copy(den_sh.at[:, pl.ds(s * NODE_CHUNK, NODE_CHUNK)], colbuf)

    def rbody(j, _):
        sl = pl.ds(j * 16, 16)
        acc = colbuf[0, sl]
        for r in range(1, 16):
            acc = acc + colbuf[r, sl]
        red_v[sl] = acc
        return 0

    lax.fori_loop(0, NODE_CHUNK // 16, rbody, 0)
    pltpu.sync_copy(red_v, den_hbm.at[pl.ds(c * NP + s * NODE_CHUNK, NODE_CHUNK)])


@functools.partial(
    pl.kernel,
    mesh=_sc_mesh,
    out_type=jax.ShapeDtypeStruct((E,), jnp.float32),
    compiler_params=pltpu.CompilerParams(needs_layout_passes=False),
    scratch_types=[
        pltpu.VMEM((NP,), jnp.float32),
        pltpu.VMEM((DCH,), jnp.float32),
        pltpu.VMEM((EPT,), jnp.float32),
        pltpu.VMEM((EPT,), jnp.int32),
    ],
)
def _coef(den_hbm, ex_hbm, dst_hbm,
          coef_hbm,
          den_v, chk_v, ex_v, dst_v):
    c = lax.axis_index("c")
    s = lax.axis_index("s")
    wid = s * 2 + c
    base = wid * EPT
    pltpu.sync_copy(den_hbm.at[pl.ds(0, NP)], den_v)
    pltpu.sync_copy(ex_hbm.at[pl.ds(base, EPT)], ex_v)
    pltpu.sync_copy(dst_hbm.at[pl.ds(base, EPT)], dst_v)
    for k in range(NP // DCH):
        pltpu.sync_copy(den_hbm.at[pl.ds(NP + k * DCH, DCH)], chk_v)

        def abody(i, _, k=k):
            sl16 = pl.ds(i * 16, 16)
            den_v[pl.ds(k * DCH + i * 16, 16)] = (
                den_v[pl.ds(k * DCH + i * 16, 16)] + chk_v[sl16])
            return 0

        lax.fori_loop(0, DCH // 16, abody, 0)

    def cbody(i, _):
        sl = pl.ds(i * 16, 16)
        dg = plsc.load_gather(den_v, [dst_v[sl]])
        ex_v[sl] = ex_v[sl] / (dg + 1e-16)
        return 0

    lax.fori_loop(0, VPT, cbody, 0)
    pltpu.sync_copy(ex_v, coef_hbm.at[pl.ds(base, EPT)])


@functools.partial(
    pl.kernel,
    mesh=_sc_mesh,
    out_type=jax.ShapeDtypeStruct((2, NP, C), jnp.float32),
    compiler_params=pltpu.CompilerParams(needs_layout_passes=False),
    scratch_types=[
        pltpu.VMEM((EPT,), jnp.float32),
        pltpu.VMEM((NB, KB), jnp.int32),
        pltpu.VMEM((NB, KB), jnp.int32),
        pltpu.VMEM((KB, C), jnp.float32),
        pltpu.VMEM_SHARED((NP, C), jnp.float32),
        pltpu.SemaphoreType.DMA,
    ],
)
def _agg(coef_hbm, src2_hbm, dst2_hbm, h_hbm,
         outp_hbm,
         coef_v, src2_v, dst2_v, rows_v, out_sh, sem):
    c = lax.axis_index("c")
    s = lax.axis_index("s")
    wid = s * 2 + c
    base = wid * EPT

    pltpu.sync_copy(coef_hbm.at[pl.ds(base, EPT)], coef_v)
    pltpu.sync_copy(src2_hbm.at[pl.ds(wid * NB, NB), :], src2_v)
    pltpu.sync_copy(dst2_hbm.at[pl.ds(wid * NB, NB), :], dst2_v)

    zeros = jnp.zeros((16,), jnp.float32)

    def zbody(i, _):
        r = i // (C // 16)
        q = i % (C // 16)
        rows_v[r, pl.ds(q * 16, 16)] = zeros
        return 0

    lax.fori_loop(0, ZROWS * (C // 16), zbody, 0)
    for t in range(NODE_CHUNK // ZROWS):
        pltpu.sync_copy(rows_v.at[pl.ds(0, ZROWS), :],
                        out_sh.at[pl.ds(s * NODE_CHUNK + t * ZROWS, ZROWS), :])
    plsc.subcore_barrier()

    def mbody(b, _):
        pltpu.async_copy(h_hbm.at[src2_v.at[b]], rows_v, sem).wait()

        def ebody(e, _):
            cvec = plsc.load_gather(coef_v, [jnp.full((16,), b * KB + e, jnp.int32)])
            for j in range(C // 16):
                sl = pl.ds(j * 16, 16)
                rows_v[e, sl] = rows_v[e, sl] * cvec
            return 0

        lax.fori_loop(0, KB, ebody, 0)
        pltpu.sync_copy(rows_v, out_sh.at[dst2_v.at[b]], add=True)
        return 0

    lax.fori_loop(0, NB, mbody, 0)
    plsc.subcore_barrier()
    pltpu.sync_copy(out_sh.at[pl.ds(s * NODE_CHUNK, NODE_CHUNK), :],
                    outp_hbm.at[c, pl.ds(s * NODE_CHUNK, NODE_CHUNK), :])


def _combine_body(p_ref, b_ref, o_ref):
    o_ref[...] = p_ref[0] + p_ref[1] + b_ref[...]


_combine = pl.pallas_call(
    _combine_body,
    grid=(125,),
    in_specs=[
        pl.BlockSpec((2, 80, C), lambda i: (0, i, 0)),
        pl.BlockSpec((1, C), lambda i: (0, 0)),
    ],
    out_specs=pl.BlockSpec((80, C), lambda i: (i, 0)),
    out_shape=jax.ShapeDtypeStruct((N, C), jnp.float32),
)


def kernel(x, edge_index, edge_attr, W, W_edge, att_src, att_dst, att_edge, bias):
    asv = att_src.reshape(1, C)
    adv = att_dst.reshape(1, C)
    aev = att_edge.reshape(1, C)
    h, a_src2, a_dst2 = _proj(x, W, asv, adv)
    a_edge = _aedge(edge_attr, W_edge, aev).reshape(E)
    src = edge_index[0]
    dst = edge_index[1]
    ex, den = _attn(a_src2.reshape(N), a_dst2.reshape(N), a_edge, src, dst)
    coef = _coef(den, ex, dst)
    outp = _agg(coef, src.reshape(NBT, KB), dst.reshape(NBT, KB), h)
    return _combine(outp, bias.reshape(1, C))


# speedup vs baseline: 20.6317x; 1.2442x over previous
import functools

import jax
import jax.numpy as jnp
from jax import lax
from jax.experimental import pallas as pl
from jax.experimental.pallas import tpu as pltpu
from jax.experimental.pallas import tpu_sc as plsc

N = 10000
E = 320000
D_IN = 128
D_EDGE = 16
C = 128
NEG_SLOPE = 0.2

NP = 10240
NTILES = 32
EPT = E // NTILES
VPT = EPT // 16
KB = 125
NB = EPT // KB
NBT = E // KB
NODE_CHUNK = NP // 16
DCH = 640
ZROWS = 80

_EBLK = 16000
_GRID_E = E // _EBLK


def _proj_body(x_ref, w_ref, asv_ref, adv_ref, h_ref, as_ref, ad_ref):
    h = lax.dot_general(x_ref[...], w_ref[...], (((1,), (1,)), ((), ())),
                        preferred_element_type=jnp.float32)
    h_ref[...] = h
    as_ref[...] = lax.dot_general(asv_ref[...], h, (((1,), (1,)), ((), ())),
                                  preferred_element_type=jnp.float32)
    ad_ref[...] = lax.dot_general(adv_ref[...], h, (((1,), (1,)), ((), ())),
                                  preferred_element_type=jnp.float32)


_proj = pl.pallas_call(
    _proj_body,
    out_shape=[
        jax.ShapeDtypeStruct((N, C), jnp.float32),
        jax.ShapeDtypeStruct((1, N), jnp.float32),
        jax.ShapeDtypeStruct((1, N), jnp.float32),
    ],
)


def _aedge_body(ea_ref, we_ref, aev_ref, out_ref):
    v = lax.dot_general(aev_ref[...], we_ref[...], (((1,), (0,)), ((), ())),
                        preferred_element_type=jnp.float32)
    ae = lax.dot_general(v, ea_ref[...], (((1,), (1,)), ((), ())),
                         preferred_element_type=jnp.float32)
    out_ref[...] = ae.reshape(1, 1, _EBLK)


_aedge = pl.pallas_call(
    _aedge_body,
    grid=(_GRID_E,),
    in_specs=[
        pl.BlockSpec((_EBLK, D_EDGE), lambda i: (i, 0)),
        pl.BlockSpec((C, D_EDGE), lambda i: (0, 0)),
        pl.BlockSpec((1, C), lambda i: (0, 0)),
    ],
    out_specs=pl.BlockSpec((1, 1, _EBLK), lambda i: (i, 0, 0)),
    out_shape=jax.ShapeDtypeStruct((_GRID_E, 1, _EBLK), jnp.float32),
)


_sc_mesh = plsc.VectorSubcoreMesh(core_axis_name="c", subcore_axis_name="s")


@functools.partial(
    pl.kernel,
    mesh=_sc_mesh,
    out_type=[
        jax.ShapeDtypeStruct((E,), jnp.float32),
        jax.ShapeDtypeStruct((2 * NP,), jnp.float32),
    ],
    compiler_params=pltpu.CompilerParams(needs_layout_passes=False),
    scratch_types=[
        pltpu.VMEM((EPT,), jnp.int32),
        pltpu.VMEM((EPT,), jnp.int32),
        pltpu.VMEM((EPT,), jnp.float32),
        pltpu.VMEM((EPT,), jnp.float32),
        pltpu.VMEM((N,), jnp.float32),
        pltpu.VMEM((N,), jnp.float32),
        pltpu.VMEM((NP,), jnp.float32),
        pltpu.VMEM((16, NODE_CHUNK), jnp.float32),
        pltpu.VMEM((NODE_CHUNK,), jnp.float32),
        pltpu.VMEM_SHARED((16, NP), jnp.float32),
    ],
)
def _attn(asrc_hbm, adst_hbm, ae_hbm, ei_hbm,
          ex_hbm, den_hbm,
          src_v, dst_v, ae_v, ex_v, asrc_v, adst_v, den_v,
          colbuf, red_v, den_sh):
    c = lax.axis_index("c")
    s = lax.axis_index("s")
    wid = s * 2 + c
    base = wid * EPT
    pltpu.sync_copy(ei_hbm.at[pl.ds(base, EPT)], src_v)
    pltpu.sync_copy(ei_hbm.at[pl.ds(E + base, EPT)], dst_v)
    pltpu.sync_copy(ae_hbm.at[pl.ds(base, EPT)], ae_v)
    pltpu.sync_copy(asrc_hbm, asrc_v)
    pltpu.sync_copy(adst_hbm, adst_v)

    zeros = jnp.zeros((16,), jnp.float32)

    def zbody(i, _):
        den_v[pl.ds(i * 16, 16)] = zeros
        return 0

    lax.fori_loop(0, NP // 16, zbody, 0)

    def body(i, _):
        sl = pl.ds(i * 16, 16)
        sv = src_v[sl]
        dv = dst_v[sl]
        a = plsc.load_gather(asrc_v, [sv]) + plsc.load_gather(adst_v, [dv]) + ae_v[sl]
        a = jnp.where(a >= 0, a, NEG_SLOPE * a)
        ev = jnp.exp(a)
        ex_v[sl] = ev
        plsc.addupdate_scatter(den_v, [dv], ev)
        return 0

    lax.fori_loop(0, VPT, body, 0)
    pltpu.sync_copy(ex_v, ex_hbm.at[pl.ds(base, EPT)])

    pltpu.sync_copy(den_v, den_sh.at[s])
    plsc.subcore_barrier()
    pltpu.sync_copy(den_sh.at[:, pl.ds(s * NODE_CHUNK, NODE_CHUNK)], colbuf)

    def rbody(j, _):
        sl = pl.ds(j * 16, 16)
        acc = colbuf[0, sl]
        for r in range(1, 16):
            acc = acc + colbuf[r, sl]
        red_v[sl] = acc
        return 0

    lax.fori_loop(0, NODE_CHUNK // 16, rbody, 0)
    pltpu.sync_copy(red_v, den_hbm.at[pl.ds(c * NP + s * NODE_CHUNK, NODE_CHUNK)])


@functools.partial(
    pl.kernel,
    mesh=_sc_mesh,
    out_type=jax.ShapeDtypeStruct((E,), jnp.float32),
    compiler_params=pltpu.CompilerParams(needs_layout_passes=False),
    scratch_types=[
        pltpu.VMEM((NP,), jnp.float32),
        pltpu.VMEM((DCH,), jnp.float32),
        pltpu.VMEM((EPT,), jnp.float32),
        pltpu.VMEM((EPT,), jnp.int32),
    ],
)
def _coef(den_hbm, ex_hbm, ei_hbm,
          coef_hbm,
          den_v, chk_v, ex_v, dst_v):
    c = lax.axis_index("c")
    s = lax.axis_index("s")
    wid = s * 2 + c
    base = wid * EPT
    pltpu.sync_copy(den_hbm.at[pl.ds(0, NP)], den_v)
    pltpu.sync_copy(ex_hbm.at[pl.ds(base, EPT)], ex_v)
    pltpu.sync_copy(ei_hbm.at[pl.ds(E + base, EPT)], dst_v)
    for k in range(NP // DCH):
        pltpu.sync_copy(den_hbm.at[pl.ds(NP + k * DCH, DCH)], chk_v)

        def abody(i, _, k=k):
            sl16 = pl.ds(i * 16, 16)
            den_v[pl.ds(k * DCH + i * 16, 16)] = (
                den_v[pl.ds(k * DCH + i * 16, 16)] + chk_v[sl16])
            return 0

        lax.fori_loop(0, DCH // 16, abody, 0)

    def cbody(i, _):
        sl = pl.ds(i * 16, 16)
        dg = plsc.load_gather(den_v, [dst_v[sl]])
        ex_v[sl] = ex_v[sl] / (dg + 1e-16)
        return 0

    lax.fori_loop(0, VPT, cbody, 0)
    pltpu.sync_copy(ex_v, coef_hbm.at[pl.ds(base, EPT)])


@functools.partial(
    pl.kernel,
    mesh=_sc_mesh,
    out_type=jax.ShapeDtypeStruct((2, NP, C), jnp.float32),
    compiler_params=pltpu.CompilerParams(needs_layout_passes=False),
    scratch_types=[
        pltpu.VMEM((EPT,), jnp.float32),
        pltpu.VMEM((NB, KB), jnp.int32),
        pltpu.VMEM((NB, KB), jnp.int32),
        pltpu.VMEM((KB, C), jnp.float32),
        pltpu.VMEM_SHARED((NP, C), jnp.float32),
        pltpu.SemaphoreType.DMA,
    ],
)
def _agg(coef_hbm, ei3_hbm, h_hbm,
         outp_hbm,
         coef_v, src2_v, dst2_v, rows_v, out_sh, sem):
    c = lax.axis_index("c")
    s = lax.axis_index("s")
    wid = s * 2 + c
    base = wid * EPT

    pltpu.sync_copy(coef_hbm.at[pl.ds(base, EPT)], coef_v)
    pltpu.sync_copy(ei3_hbm.at[0, pl.ds(wid * NB, NB), :], src2_v)
    pltpu.sync_copy(ei3_hbm.at[1, pl.ds(wid * NB, NB), :], dst2_v)

    zeros = jnp.zeros((16,), jnp.float32)

    def zbody(i, _):
        r = i // (C // 16)
        q = i % (C // 16)
        rows_v[r, pl.ds(q * 16, 16)] = zeros
        return 0

    lax.fori_loop(0, ZROWS * (C // 16), zbody, 0)
    for t in range(NODE_CHUNK // ZROWS):
        pltpu.sync_copy(rows_v.at[pl.ds(0, ZROWS), :],
                        out_sh.at[pl.ds(s * NODE_CHUNK + t * ZROWS, ZROWS), :])
    plsc.subcore_barrier()

    def mbody(b, _):
        pltpu.async_copy(h_hbm.at[src2_v.at[b]], rows_v, sem).wait()

        def ebody(e, _):
            cvec = plsc.load_gather(coef_v, [jnp.full((16,), b * KB + e, jnp.int32)])
            for j in range(C // 16):
                sl = pl.ds(j * 16, 16)
                rows_v[e, sl] = rows_v[e, sl] * cvec
            return 0

        lax.fori_loop(0, KB, ebody, 0)
        pltpu.sync_copy(rows_v, out_sh.at[dst2_v.at[b]], add=True)
        return 0

    lax.fori_loop(0, NB, mbody, 0)
    plsc.subcore_barrier()
    pltpu.sync_copy(out_sh.at[pl.ds(s * NODE_CHUNK, NODE_CHUNK), :],
                    outp_hbm.at[c, pl.ds(s * NODE_CHUNK, NODE_CHUNK), :])


def _combine_body(p_ref, b_ref, o_ref):
    o_ref[...] = p_ref[0] + p_ref[1] + b_ref[...]


_combine = pl.pallas_call(
    _combine_body,
    grid=(10,),
    in_specs=[
        pl.BlockSpec((2, 1000, C), lambda i: (0, i, 0)),
        pl.BlockSpec((1, C), lambda i: (0, 0)),
    ],
    out_specs=pl.BlockSpec((1000, C), lambda i: (i, 0)),
    out_shape=jax.ShapeDtypeStruct((N, C), jnp.float32),
)


def kernel(x, edge_index, edge_attr, W, W_edge, att_src, att_dst, att_edge, bias):
    asv = att_src.reshape(1, C)
    adv = att_dst.reshape(1, C)
    aev = att_edge.reshape(1, C)
    h, a_src2, a_dst2 = _proj(x, W, asv, adv)
    a_edge = _aedge(edge_attr, W_edge, aev).reshape(E)
    eflat = edge_index.reshape(2 * E)
    ei3 = edge_index.reshape(2, NBT, KB)
    ex, den = _attn(a_src2.reshape(N), a_dst2.reshape(N), a_edge, eflat)
    coef = _coef(den, ex, eflat)
    outp = _agg(coef, ei3, h)
    return _combine(outp, bias.reshape(1, C))


# speedup vs baseline: 26.5500x; 1.2869x over previous
import functools

import jax
import jax.numpy as jnp
from jax import lax
from jax.experimental import pallas as pl
from jax.experimental.pallas import tpu as pltpu
from jax.experimental.pallas import tpu_sc as plsc

N = 10000
E = 320000
D_IN = 128
D_EDGE = 16
C = 128
NEG_SLOPE = 0.2

NP = 10240
NTILES = 32
EPT = E // NTILES
VPT = EPT // 16
KB = 125
NB = EPT // KB
NBT = E // KB
NODE_CHUNK = NP // 16
DCH = 640
ZROWS = 40

_EBLK = 16000
_GRID_E = E // _EBLK


def _proj_body(x_ref, w_ref, asv_ref, adv_ref, h_ref, as_ref, ad_ref):
    h = lax.dot_general(x_ref[...], w_ref[...], (((1,), (1,)), ((), ())),
                        preferred_element_type=jnp.float32)
    h_ref[...] = h
    as_ref[...] = lax.dot_general(asv_ref[...], h, (((1,), (1,)), ((), ())),
                                  preferred_element_type=jnp.float32)
    ad_ref[...] = lax.dot_general(adv_ref[...], h, (((1,), (1,)), ((), ())),
                                  preferred_element_type=jnp.float32)


_proj = pl.pallas_call(
    _proj_body,
    out_shape=[
        jax.ShapeDtypeStruct((N, C), jnp.float32),
        jax.ShapeDtypeStruct((1, N), jnp.float32),
        jax.ShapeDtypeStruct((1, N), jnp.float32),
    ],
)


def _aedge_body(eat_ref, we_ref, aev_ref, out_ref):
    v = lax.dot_general(aev_ref[...], we_ref[...], (((1,), (0,)), ((), ())),
                        preferred_element_type=jnp.float32)
    ae = lax.dot_general(v, eat_ref[...], (((1,), (0,)), ((), ())),
                         preferred_element_type=jnp.float32)
    out_ref[...] = ae.reshape(1, 1, _EBLK)


_aedge = pl.pallas_call(
    _aedge_body,
    grid=(_GRID_E,),
    in_specs=[
        pl.BlockSpec((D_EDGE, _EBLK), lambda i: (0, i)),
        pl.BlockSpec((C, D_EDGE), lambda i: (0, 0)),
        pl.BlockSpec((1, C), lambda i: (0, 0)),
    ],
    out_specs=pl.BlockSpec((1, 1, _EBLK), lambda i: (i, 0, 0)),
    out_shape=jax.ShapeDtypeStruct((_GRID_E, 1, _EBLK), jnp.float32),
)


_sc_mesh = plsc.VectorSubcoreMesh(core_axis_name="c", subcore_axis_name="s")


@functools.partial(
    pl.kernel,
    mesh=_sc_mesh,
    out_type=[
        jax.ShapeDtypeStruct((E,), jnp.float32),
        jax.ShapeDtypeStruct((2 * NP,), jnp.float32),
    ],
    compiler_params=pltpu.CompilerParams(needs_layout_passes=False),
    scratch_types=[
        pltpu.VMEM((EPT,), jnp.int32),
        pltpu.VMEM((EPT,), jnp.int32),
        pltpu.VMEM((EPT,), jnp.float32),
        pltpu.VMEM((EPT,), jnp.float32),
        pltpu.VMEM((N,), jnp.float32),
        pltpu.VMEM((N,), jnp.float32),
        pltpu.VMEM((NP,), jnp.float32),
        pltpu.VMEM((16, NODE_CHUNK), jnp.float32),
        pltpu.VMEM((NODE_CHUNK,), jnp.float32),
        pltpu.VMEM_SHARED((16, NP), jnp.float32),
    ],
)
def _attn(asrc_hbm, adst_hbm, ae_hbm, ei_hbm,
          ex_hbm, den_hbm,
          src_v, dst_v, ae_v, ex_v, asrc_v, adst_v, den_v,
          colbuf, red_v, den_sh):
    c = lax.axis_index("c")
    s = lax.axis_index("s")
    wid = s * 2 + c
    base = wid * EPT
    pltpu.sync_copy(ei_hbm.at[pl.ds(base, EPT)], src_v)
    pltpu.sync_copy(ei_hbm.at[pl.ds(E + base, EPT)], dst_v)
    pltpu.sync_copy(ae_hbm.at[pl.ds(base, EPT)], ae_v)
    pltpu.sync_copy(asrc_hbm, asrc_v)
    pltpu.sync_copy(adst_hbm, adst_v)

    zeros = jnp.zeros((16,), jnp.float32)

    def zbody(i, _):
        den_v[pl.ds(i * 16, 16)] = zeros
        return 0

    lax.fori_loop(0, NP // 16, zbody, 0)

    def body(i, _):
        sl = pl.ds(i * 16, 16)
        sv = src_v[sl]
        dv = dst_v[sl]
        a = plsc.load_gather(asrc_v, [sv]) + plsc.load_gather(adst_v, [dv]) + ae_v[sl]
        a = jnp.where(a >= 0, a, NEG_SLOPE * a)
        ev = jnp.exp(a)
        ex_v[sl] = ev
        plsc.addupdate_scatter(den_v, [dv], ev)
        return 0

    lax.fori_loop(0, VPT, body, 0)
    pltpu.sync_copy(ex_v, ex_hbm.at[pl.ds(base, EPT)])

    pltpu.sync_copy(den_v, den_sh.at[s])
    plsc.subcore_barrier()
    pltpu.sync_copy(den_sh.at[:, pl.ds(s * NODE_CHUNK, NODE_CHUNK)], colbuf)

    def rbody(j, _):
        sl = pl.ds(j * 16, 16)
        acc = colbuf[0, sl]
        for r in range(1, 16):
            acc = acc + colbuf[r, sl]
        red_v[sl] = acc
        return 0

    lax.fori_loop(0, NODE_CHUNK // 16, rbody, 0)
    pltpu.sync_copy(red_v, den_hbm.at[pl.ds(c * NP + s * NODE_CHUNK, NODE_CHUNK)])


@functools.partial(
    pl.kernel,
    mesh=_sc_mesh,
    out_type=jax.ShapeDtypeStruct((E,), jnp.float32),
    compiler_params=pltpu.CompilerParams(needs_layout_passes=False),
    scratch_types=[
        pltpu.VMEM((NP,), jnp.float32),
        pltpu.VMEM((DCH,), jnp.float32),
        pltpu.VMEM((EPT,), jnp.float32),
        pltpu.VMEM((EPT,), jnp.int32),
    ],
)
def _coef(den_hbm, ex_hbm, ei_hbm,
          coef_hbm,
          den_v, chk_v, ex_v, dst_v):
    c = lax.axis_index("c")
    s = lax.axis_index("s")
    wid = s * 2 + c
    base = wid * EPT
    pltpu.sync_copy(den_hbm.at[pl.ds(0, NP)], den_v)
    pltpu.sync_copy(ex_hbm.at[pl.ds(base, EPT)], ex_v)
    pltpu.sync_copy(ei_hbm.at[pl.ds(E + base, EPT)], dst_v)
    for k in range(NP // DCH):
        pltpu.sync_copy(den_hbm.at[pl.ds(NP + k * DCH, DCH)], chk_v)

        def abody(i, _, k=k):
            sl16 = pl.ds(i * 16, 16)
            den_v[pl.ds(k * DCH + i * 16, 16)] = (
                den_v[pl.ds(k * DCH + i * 16, 16)] + chk_v[sl16])
            return 0

        lax.fori_loop(0, DCH // 16, abody, 0)

    def cbody(i, _):
        sl = pl.ds(i * 16, 16)
        dg = plsc.load_gather(den_v, [dst_v[sl]])
        ex_v[sl] = ex_v[sl] / (dg + 1e-16)
        return 0

    lax.fori_loop(0, VPT, cbody, 0)
    pltpu.sync_copy(ex_v, coef_hbm.at[pl.ds(base, EPT)])


@functools.partial(
    pl.kernel,
    mesh=_sc_mesh,
    out_type=jax.ShapeDtypeStruct((2, NP, C), jnp.float32),
    compiler_params=pltpu.CompilerParams(needs_layout_passes=False),
    scratch_types=[
        pltpu.VMEM((EPT,), jnp.float32),
        pltpu.VMEM((NB, KB), jnp.int32),
        pltpu.VMEM((NB, KB), jnp.int32),
        pltpu.VMEM((KB, C), jnp.float32),
        pltpu.VMEM_SHARED((NP, C), jnp.float32),
        pltpu.SemaphoreType.DMA,
    ],
)
def _agg(coef_hbm, ei3_hbm, h_hbm,
         outp_hbm,
         coef_v, src2_v, dst2_v, rows0_v, out_sh, sem0):
    c = lax.axis_index("c")
    s = lax.axis_index("s")
    wid = s * 2 + c
    base = wid * EPT

    pltpu.sync_copy(coef_hbm.at[pl.ds(base, EPT)], coef_v)
    pltpu.sync_copy(ei3_hbm.at[0, pl.ds(wid * NB, NB), :], src2_v)
    pltpu.sync_copy(ei3_hbm.at[1, pl.ds(wid * NB, NB), :], dst2_v)

    zeros = jnp.zeros((16,), jnp.float32)

    def zbody(i, _):
        r = i // (C // 16)
        q = i % (C // 16)
        rows0_v[r, pl.ds(q * 16, 16)] = zeros
        return 0

    lax.fori_loop(0, ZROWS * (C // 16), zbody, 0)
    for t in range(NODE_CHUNK // ZROWS):
        pltpu.sync_copy(rows0_v.at[pl.ds(0, ZROWS), :],
                        out_sh.at[pl.ds(s * NODE_CHUNK + t * ZROWS, ZROWS), :])
    plsc.subcore_barrier()

    def mbody(b, _):
        pltpu.async_copy(h_hbm.at[src2_v.at[b]], rows0_v, sem0).wait()

        def ebody(e, _):
            cvec = plsc.load_gather(coef_v, [jnp.full((16,), b * KB + e, jnp.int32)])
            for j in range(C // 16):
                sl = pl.ds(j * 16, 16)
                rows0_v[e, sl] = rows0_v[e, sl] * cvec
            return 0

        lax.fori_loop(0, KB, ebody, 0)
        pltpu.sync_copy(rows0_v, out_sh.at[dst2_v.at[b]], add=True)
        return 0

    lax.fori_loop(0, NB, mbody, 0)
    plsc.subcore_barrier()
    pltpu.sync_copy(out_sh.at[pl.ds(s * NODE_CHUNK, NODE_CHUNK), :],
                    outp_hbm.at[c, pl.ds(s * NODE_CHUNK, NODE_CHUNK), :])


def _combine_body(p_ref, b_ref, o_ref):
    o_ref[...] = p_ref[0] + p_ref[1] + b_ref[...]


_combine = pl.pallas_call(
    _combine_body,
    grid=(10,),
    in_specs=[
        pl.BlockSpec((2, 1000, C), lambda i: (0, i, 0)),
        pl.BlockSpec((1, C), lambda i: (0, 0)),
    ],
    out_specs=pl.BlockSpec((1000, C), lambda i: (i, 0)),
    out_shape=jax.ShapeDtypeStruct((N, C), jnp.float32),
)


def kernel(x, edge_index, edge_attr, W, W_edge, att_src, att_dst, att_edge, bias):
    asv = att_src.reshape(1, C)
    adv = att_dst.reshape(1, C)
    aev = att_edge.reshape(1, C)
    h, a_src2, a_dst2 = _proj(x, W, asv, adv)
    a_edge = _aedge(edge_attr.T, W_edge, aev).reshape(E)
    eflat = edge_index.reshape(2 * E)
    ei3 = edge_index.reshape(2, NBT, KB)
    ex, den = _attn(a_src2.reshape(N), a_dst2.reshape(N), a_edge, eflat)
    coef = _coef(den, ex, eflat)
    outp = _agg(coef, ei3, h)
    return _combine(outp, bias.reshape(1, C))


# speedup vs baseline: 31.9188x; 1.2022x over previous
import functools

import jax
import jax.numpy as jnp
from jax import lax
from jax.experimental import pallas as pl
from jax.experimental.pallas import tpu as pltpu
from jax.experimental.pallas import tpu_sc as plsc

N = 10000
E = 320000
D_IN = 128
D_EDGE = 16
C = 128
NEG_SLOPE = 0.2

NP = 10240
NTILES = 32
EPT = E // NTILES
VPT = EPT // 16
KB = 125
NB = EPT // KB
NBT = E // KB
NODE_CHUNK = NP // 16
DCH = 640
ZROWS = 40

_EBLK = 16000
_GRID_E = E // _EBLK


def _proj_body(x_ref, w_ref, asv_ref, adv_ref, h_ref, as_ref, ad_ref):
    h = lax.dot_general(x_ref[...], w_ref[...], (((1,), (1,)), ((), ())),
                        preferred_element_type=jnp.float32)
    h_ref[...] = h
    as_ref[...] = lax.dot_general(asv_ref[...], h, (((1,), (1,)), ((), ())),
                                  preferred_element_type=jnp.float32)
    ad_ref[...] = lax.dot_general(adv_ref[...], h, (((1,), (1,)), ((), ())),
                                  preferred_element_type=jnp.float32)


_proj = pl.pallas_call(
    _proj_body,
    out_shape=[
        jax.ShapeDtypeStruct((N, C), jnp.float32),
        jax.ShapeDtypeStruct((1, N), jnp.float32),
        jax.ShapeDtypeStruct((1, N), jnp.float32),
    ],
)


def _aedge_body(eat_ref, we_ref, aev_ref, out_ref):
    v = lax.dot_general(aev_ref[...], we_ref[...], (((1,), (0,)), ((), ())),
                        preferred_element_type=jnp.float32)
    ae = lax.dot_general(v, eat_ref[...], (((1,), (0,)), ((), ())),
                         preferred_element_type=jnp.float32)
    out_ref[...] = ae.reshape(1, 1, _EBLK)


_aedge = pl.pallas_call(
    _aedge_body,
    grid=(_GRID_E,),
    in_specs=[
        pl.BlockSpec((D_EDGE, _EBLK), lambda i: (0, i)),
        pl.BlockSpec((C, D_EDGE), lambda i: (0, 0)),
        pl.BlockSpec((1, C), lambda i: (0, 0)),
    ],
    out_specs=pl.BlockSpec((1, 1, _EBLK), lambda i: (i, 0, 0)),
    out_shape=jax.ShapeDtypeStruct((_GRID_E, 1, _EBLK), jnp.float32),
)


_sc_mesh = plsc.VectorSubcoreMesh(core_axis_name="c", subcore_axis_name="s")


@functools.partial(
    pl.kernel,
    mesh=_sc_mesh,
    out_type=[
        jax.ShapeDtypeStruct((E,), jnp.float32),
        jax.ShapeDtypeStruct((2 * NP,), jnp.float32),
    ],
    compiler_params=pltpu.CompilerParams(needs_layout_passes=False),
    scratch_types=[
        pltpu.VMEM((EPT,), jnp.int32),
        pltpu.VMEM((EPT,), jnp.int32),
        pltpu.VMEM((EPT,), jnp.float32),
        pltpu.VMEM((EPT,), jnp.float32),
        pltpu.VMEM((N,), jnp.float32),
        pltpu.VMEM((N,), jnp.float32),
        pltpu.VMEM((NP,), jnp.float32),
        pltpu.VMEM((16, NODE_CHUNK), jnp.float32),
        pltpu.VMEM((NODE_CHUNK,), jnp.float32),
        pltpu.VMEM_SHARED((16, NP), jnp.float32),
    ],
)
def _attn(asrc_hbm, adst_hbm, ae_hbm, ei_hbm,
          ex_hbm, den_hbm,
          src_v, dst_v, ae_v, ex_v, asrc_v, adst_v, den_v,
          colbuf, red_v, den_sh):
    c = lax.axis_index("c")
    s = lax.axis_index("s")
    wid = s * 2 + c
    base = wid * EPT
    pltpu.sync_copy(ei_hbm.at[pl.ds(base, EPT)], src_v)
    pltpu.sync_copy(ei_hbm.at[pl.ds(E + base, EPT)], dst_v)
    pltpu.sync_copy(ae_hbm.at[pl.ds(base, EPT)], ae_v)
    pltpu.sync_copy(asrc_hbm, asrc_v)
    pltpu.sync_copy(adst_hbm, adst_v)

    zeros = jnp.zeros((16,), jnp.float32)

    def zbody(i, _):
        den_v[pl.ds(i * 16, 16)] = zeros
        return 0

    lax.fori_loop(0, NP // 16, zbody, 0)

    def body(i, _):
        sl = pl.ds(i * 16, 16)
        sv = src_v[sl]
        dv = dst_v[sl]
        a = plsc.load_gather(asrc_v, [sv]) + plsc.load_gather(adst_v, [dv]) + ae_v[sl]
        a = jnp.where(a >= 0, a, NEG_SLOPE * a)
        ev = jnp.exp(a)
        ex_v[sl] = ev
        plsc.addupdate_scatter(den_v, [dv], ev)
        return 0

    lax.fori_loop(0, VPT, body, 0)
    pltpu.sync_copy(ex_v, ex_hbm.at[pl.ds(base, EPT)])

    pltpu.sync_copy(den_v, den_sh.at[s])
    plsc.subcore_barrier()
    pltpu.sync_copy(den_sh.at[:, pl.ds(s * NODE_CHUNK, NODE_CHUNK)], colbuf)

    def rbody(j, _):
        sl = pl.ds(j * 16, 16)
        acc = colbuf[0, sl]
        for r in range(1, 16):
            acc = acc + colbuf[r, sl]
        red_v[sl] = acc
        return 0

    lax.fori_loop(0, NODE_CHUNK // 16, rbody, 0)
    pltpu.sync_copy(red_v, den_hbm.at[pl.ds(c * NP + s * NODE_CHUNK, NODE_CHUNK)])


@functools.partial(
    pl.kernel,
    mesh=_sc_mesh,
    out_type=jax.ShapeDtypeStruct((E,), jnp.float32),
    compiler_params=pltpu.CompilerParams(needs_layout_passes=False),
    scratch_types=[
        pltpu.VMEM((NP,), jnp.float32),
        pltpu.VMEM((DCH,), jnp.float32),
        pltpu.VMEM((EPT,), jnp.float32),
        pltpu.VMEM((EPT,), jnp.int32),
    ],
)
def _coef(den_hbm, ex_hbm, ei_hbm,
          coef_hbm,
          den_v, chk_v, ex_v, dst_v):
    c = lax.axis_index("c")
    s = lax.axis_index("s")
    wid = s * 2 + c
    base = wid * EPT
    pltpu.sync_copy(den_hbm.at[pl.ds(0, NP)], den_v)
    pltpu.sync_copy(ex_hbm.at[pl.ds(base, EPT)], ex_v)
    pltpu.sync_copy(ei_hbm.at[pl.ds(E + base, EPT)], dst_v)
    for k in range(NP // DCH):
        pltpu.sync_copy(den_hbm.at[pl.ds(NP + k * DCH, DCH)], chk_v)

        def abody(i, _, k=k):
            sl16 = pl.ds(i * 16, 16)
            den_v[pl.ds(k * DCH + i * 16, 16)] = (
                den_v[pl.ds(k * DCH + i * 16, 16)] + chk_v[sl16])
            return 0

        lax.fori_loop(0, DCH // 16, abody, 0)

    def cbody(i, _):
        sl = pl.ds(i * 16, 16)
        dg = plsc.load_gather(den_v, [dst_v[sl]])
        ex_v[sl] = ex_v[sl] / (dg + 1e-16)
        return 0

    lax.fori_loop(0, VPT, cbody, 0)
    pltpu.sync_copy(ex_v, coef_hbm.at[pl.ds(base, EPT)])


@functools.partial(
    pl.kernel,
    mesh=_sc_mesh,
    out_type=jax.ShapeDtypeStruct((2, NP, C), jnp.float32),
    compiler_params=pltpu.CompilerParams(needs_layout_passes=False),
    scratch_types=[
        pltpu.VMEM((EPT,), jnp.float32),
        pltpu.VMEM((EPT,), jnp.int32),
        pltpu.VMEM((EPT,), jnp.int32),
        [pltpu.VMEM((16, C), jnp.float32)] * 5,
        [pltpu.SemaphoreType.DMA] * 5,
        pltpu.VMEM_SHARED((NP, C), jnp.float32),
    ],
)
def _agg(coef_hbm, eflat_hbm, h_hbm,
         outp_hbm,
         coef_v, src_v, dst_v, bufs, sems, out_sh):
    c = lax.axis_index("c")
    s = lax.axis_index("s")
    wid = s * 2 + c
    base = wid * EPT

    pltpu.sync_copy(coef_hbm.at[pl.ds(base, EPT)], coef_v)
    pltpu.sync_copy(eflat_hbm.at[pl.ds(base, EPT)], src_v)
    pltpu.sync_copy(eflat_hbm.at[pl.ds(E + base, EPT)], dst_v)

    zeros = jnp.zeros((16,), jnp.float32)

    def zbody(i, _):
        r = i // (C // 16)
        q = i % (C // 16)
        bufs[0][r, pl.ds(q * 16, 16)] = zeros
        return 0

    lax.fori_loop(0, 16 * (C // 16), zbody, 0)
    for t in range(NODE_CHUNK // 16):
        pltpu.sync_copy(bufs[0],
                        out_sh.at[pl.ds(s * NODE_CHUNK + t * 16, 16), :])
    plsc.subcore_barrier()

    NMB = EPT // 16
    for k in range(5):
        pltpu.async_copy(h_hbm.at[src_v[pl.ds(k * 16, 16)]], bufs[k], sems[k])

    def mbody(g, _):
        m0 = g * 5
        for k in range(5):
            m = m0 + k
            sl = pl.ds(m * 16, 16)
            pltpu.make_async_copy(h_hbm.at[src_v[sl]], bufs[k], sems[k]).wait()

            def ebody(e, _, k=k, m=m):
                cvec = plsc.load_gather(
                    coef_v, [jnp.full((16,), m * 16 + e, jnp.int32)])
                for j in range(C // 16):
                    cl = pl.ds(j * 16, 16)
                    bufs[k][e, cl] = bufs[k][e, cl] * cvec
                return 0

            lax.fori_loop(0, 16, ebody, 0)
            pltpu.sync_copy(bufs[k], out_sh.at[dst_v[sl]], add=True)

            @pl.when(m + 5 < NMB)
            def _(k=k, m=m):
                nsl = pl.ds((m + 5) * 16, 16)
                pltpu.async_copy(h_hbm.at[src_v[nsl]], bufs[k], sems[k])
        return 0

    lax.fori_loop(0, NMB // 5, mbody, 0)
    plsc.subcore_barrier()
    pltpu.sync_copy(out_sh.at[pl.ds(s * NODE_CHUNK, NODE_CHUNK), :],
                    outp_hbm.at[c, pl.ds(s * NODE_CHUNK, NODE_CHUNK), :])


def _combine_body(p_ref, b_ref, o_ref):
    o_ref[...] = p_ref[0] + p_ref[1] + b_ref[...]


_combine = pl.pallas_call(
    _combine_body,
    grid=(10,),
    in_specs=[
        pl.BlockSpec((2, 1000, C), lambda i: (0, i, 0)),
        pl.BlockSpec((1, C), lambda i: (0, 0)),
    ],
    out_specs=pl.BlockSpec((1000, C), lambda i: (i, 0)),
    out_shape=jax.ShapeDtypeStruct((N, C), jnp.float32),
)


def kernel(x, edge_index, edge_attr, W, W_edge, att_src, att_dst, att_edge, bias):
    asv = att_src.reshape(1, C)
    adv = att_dst.reshape(1, C)
    aev = att_edge.reshape(1, C)
    h, a_src2, a_dst2 = _proj(x, W, asv, adv)
    a_edge = _aedge(edge_attr.T, W_edge, aev).reshape(E)
    eflat = edge_index.reshape(2 * E)
    ex, den = _attn(a_src2.reshape(N), a_dst2.reshape(N), a_edge, eflat)
    coef = _coef(den, ex, eflat)
    outp = _agg(coef, eflat, h)
    return _combine(outp, bias.reshape(1, C))


# speedup vs baseline: 31.9230x; 1.0001x over previous
import functools

import jax
import jax.numpy as jnp
from jax import lax
from jax.experimental import pallas as pl
from jax.experimental.pallas import tpu as pltpu
from jax.experimental.pallas import tpu_sc as plsc

N = 10000
E = 320000
D_IN = 128
D_EDGE = 16
C = 128
NEG_SLOPE = 0.2

NP = 10240
NTILES = 32
EPT = E // NTILES
VPT = EPT // 16
NODE_CHUNK = NP // 16
DCH = 640

_EBLK = 16000
_GRID_E = E // _EBLK


def _proj_body(x_ref, w_ref, asv_ref, adv_ref, h_ref, as_ref, ad_ref):
    h = lax.dot_general(x_ref[...], w_ref[...], (((1,), (1,)), ((), ())),
                        preferred_element_type=jnp.float32)
    h_ref[...] = h
    as_ref[...] = lax.dot_general(asv_ref[...], h, (((1,), (1,)), ((), ())),
                                  preferred_element_type=jnp.float32)
    ad_ref[...] = lax.dot_general(adv_ref[...], h, (((1,), (1,)), ((), ())),
                                  preferred_element_type=jnp.float32)


_proj = pl.pallas_call(
    _proj_body,
    out_shape=[
        jax.ShapeDtypeStruct((N, C), jnp.float32),
        jax.ShapeDtypeStruct((1, N), jnp.float32),
        jax.ShapeDtypeStruct((1, N), jnp.float32),
    ],
)


def _aedge_body(eat_ref, we_ref, aev_ref, out_ref):
    v = lax.dot_general(aev_ref[...], we_ref[...], (((1,), (0,)), ((), ())),
                        preferred_element_type=jnp.float32)
    ae = lax.dot_general(v, eat_ref[...], (((1,), (0,)), ((), ())),
                         preferred_element_type=jnp.float32)
    out_ref[...] = ae.reshape(1, 1, _EBLK)


_aedge = pl.pallas_call(
    _aedge_body,
    grid=(_GRID_E,),
    in_specs=[
        pl.BlockSpec((D_EDGE, _EBLK), lambda i: (0, i)),
        pl.BlockSpec((C, D_EDGE), lambda i: (0, 0)),
        pl.BlockSpec((1, C), lambda i: (0, 0)),
    ],
    out_specs=pl.BlockSpec((1, 1, _EBLK), lambda i: (i, 0, 0)),
    out_shape=jax.ShapeDtypeStruct((_GRID_E, 1, _EBLK), jnp.float32),
)


_sc_mesh = plsc.VectorSubcoreMesh(core_axis_name="c", subcore_axis_name="s")


@functools.partial(
    pl.kernel,
    mesh=_sc_mesh,
    out_type=[
        jax.ShapeDtypeStruct((E,), jnp.float32),
        jax.ShapeDtypeStruct((2 * NP,), jnp.float32),
    ],
    compiler_params=pltpu.CompilerParams(needs_layout_passes=False),
    scratch_types=[
        pltpu.VMEM((EPT,), jnp.int32),
        pltpu.VMEM((EPT,), jnp.int32),
        pltpu.VMEM((EPT,), jnp.float32),
        pltpu.VMEM((EPT,), jnp.float32),
        pltpu.VMEM((N,), jnp.float32),
        pltpu.VMEM((N,), jnp.float32),
        pltpu.VMEM((NP,), jnp.float32),
        pltpu.VMEM((16, NODE_CHUNK), jnp.float32),
        pltpu.VMEM((NODE_CHUNK,), jnp.float32),
        pltpu.VMEM_SHARED((16, NP), jnp.float32),
    ],
)
def _attn(asrc_hbm, adst_hbm, ae_hbm, ei_hbm,
          ex_hbm, den_hbm,
          src_v, dst_v, ae_v, ex_v, asrc_v, adst_v, den_v,
          colbuf, red_v, den_sh):
    c = lax.axis_index("c")
    s = lax.axis_index("s")
    wid = s * 2 + c
    base = wid * EPT
    pltpu.sync_copy(ei_hbm.at[pl.ds(base, EPT)], src_v)
    pltpu.sync_copy(ei_hbm.at[pl.ds(E + base, EPT)], dst_v)
    pltpu.sync_copy(ae_hbm.at[pl.ds(base, EPT)], ae_v)
    pltpu.sync_copy(asrc_hbm, asrc_v)
    pltpu.sync_copy(adst_hbm, adst_v)

    zeros = jnp.zeros((16,), jnp.float32)

    def zbody(i, _):
        den_v[pl.ds(i * 16, 16)] = zeros
        return 0

    lax.fori_loop(0, NP // 16, zbody, 0)

    def body(i, _):
        sl = pl.ds(i * 16, 16)
        sv = src_v[sl]
        dv = dst_v[sl]
        a = plsc.load_gather(asrc_v, [sv]) + plsc.load_gather(adst_v, [dv]) + ae_v[sl]
        a = jnp.where(a >= 0, a, NEG_SLOPE * a)
        ev = jnp.exp(a)
        ex_v[sl] = ev
        plsc.addupdate_scatter(den_v, [dv], ev)
        return 0

    lax.fori_loop(0, VPT, body, 0)
    pltpu.sync_copy(ex_v, ex_hbm.at[pl.ds(base, EPT)])

    pltpu.sync_copy(den_v, den_sh.at[s])
    plsc.subcore_barrier()
    pltpu.sync_copy(den_sh.at[:, pl.ds(s * NODE_CHUNK, NODE_CHUNK)], colbuf)

    def rbody(j, _):
        sl = pl.ds(j * 16, 16)
        acc = colbuf[0, sl]
        for r in range(1, 16):
            acc = acc + colbuf[r, sl]
        red_v[sl] = acc
        return 0

    lax.fori_loop(0, NODE_CHUNK // 16, rbody, 0)
    pltpu.sync_copy(red_v, den_hbm.at[pl.ds(c * NP + s * NODE_CHUNK, NODE_CHUNK)])


@functools.partial(
    pl.kernel,
    mesh=_sc_mesh,
    out_type=jax.ShapeDtypeStruct((E,), jnp.float32),
    compiler_params=pltpu.CompilerParams(needs_layout_passes=False),
    scratch_types=[
        pltpu.VMEM((NP,), jnp.float32),
        pltpu.VMEM((DCH,), jnp.float32),
        pltpu.VMEM((EPT,), jnp.float32),
        pltpu.VMEM((EPT,), jnp.int32),
    ],
)
def _coef(den_hbm, ex_hbm, ei_hbm,
          coef_hbm,
          den_v, chk_v, ex_v, dst_v):
    c = lax.axis_index("c")
    s = lax.axis_index("s")
    wid = s * 2 + c
    base = wid * EPT
    pltpu.sync_copy(den_hbm.at[pl.ds(0, NP)], den_v)
    pltpu.sync_copy(ex_hbm.at[pl.ds(base, EPT)], ex_v)
    pltpu.sync_copy(ei_hbm.at[pl.ds(E + base, EPT)], dst_v)
    for k in range(NP // DCH):
        pltpu.sync_copy(den_hbm.at[pl.ds(NP + k * DCH, DCH)], chk_v)

        def abody(i, _, k=k):
            sl16 = pl.ds(i * 16, 16)
            den_v[pl.ds(k * DCH + i * 16, 16)] = (
                den_v[pl.ds(k * DCH + i * 16, 16)] + chk_v[sl16])
            return 0

        lax.fori_loop(0, DCH // 16, abody, 0)

    def cbody(i, _):
        sl = pl.ds(i * 16, 16)
        dg = plsc.load_gather(den_v, [dst_v[sl]])
        ex_v[sl] = ex_v[sl] / (dg + 1e-16)
        return 0

    lax.fori_loop(0, VPT, cbody, 0)
    pltpu.sync_copy(ex_v, coef_hbm.at[pl.ds(base, EPT)])


@functools.partial(
    pl.kernel,
    mesh=_sc_mesh,
    out_type=jax.ShapeDtypeStruct((2, NP, C), jnp.float32),
    compiler_params=pltpu.CompilerParams(needs_layout_passes=False),
    scratch_types=[
        pltpu.VMEM((EPT,), jnp.float32),
        pltpu.VMEM((EPT,), jnp.int32),
        pltpu.VMEM((EPT,), jnp.int32),
        [pltpu.VMEM((16, C), jnp.float32)] * 5,
        [pltpu.SemaphoreType.DMA] * 5,
        pltpu.VMEM_SHARED((NP, C), jnp.float32),
    ],
)
def _agg(coef_hbm, eflat_hbm, h_hbm,
         outp_hbm,
         coef_v, src_v, dst_v, bufs, sems, out_sh):
    c = lax.axis_index("c")
    s = lax.axis_index("s")
    wid = s * 2 + c
    base = wid * EPT

    pltpu.sync_copy(coef_hbm.at[pl.ds(base, EPT)], coef_v)
    pltpu.sync_copy(eflat_hbm.at[pl.ds(base, EPT)], src_v)
    pltpu.sync_copy(eflat_hbm.at[pl.ds(E + base, EPT)], dst_v)

    zeros = jnp.zeros((16,), jnp.float32)

    def zbody(i, _):
        r = i // (C // 16)
        q = i % (C // 16)
        bufs[0][r, pl.ds(q * 16, 16)] = zeros
        return 0

    lax.fori_loop(0, 16 * (C // 16), zbody, 0)
    for t in range(NODE_CHUNK // 16):
        pltpu.sync_copy(bufs[0],
                        out_sh.at[pl.ds(s * NODE_CHUNK + t * 16, 16), :])
    plsc.subcore_barrier()

    NMB = EPT // 16
    for k in range(5):
        pltpu.async_copy(h_hbm.at[src_v[pl.ds(k * 16, 16)]], bufs[k], sems[k])

    def mbody(g, _):
        m0 = g * 5
        for k in range(5):
            m = m0 + k
            sl = pl.ds(m * 16, 16)
            pltpu.make_async_copy(h_hbm.at[src_v[sl]], bufs[k], sems[k]).wait()

            def ebody(e, _, k=k, m=m):
                cvec = plsc.load_gather(
                    coef_v, [jnp.full((16,), m * 16 + e, jnp.int32)])
                for j in range(C // 16):
                    cl = pl.ds(j * 16, 16)
                    bufs[k][e, cl] = bufs[k][e, cl] * cvec
                return 0

            lax.fori_loop(0, 16, ebody, 0)
            pltpu.sync_copy(bufs[k], out_sh.at[dst_v[sl]], add=True)

            @pl.when(m + 5 < NMB)
            def _(k=k, m=m):
                nsl = pl.ds((m + 5) * 16, 16)
                pltpu.async_copy(h_hbm.at[src_v[nsl]], bufs[k], sems[k])
        return 0

    lax.fori_loop(0, NMB // 5, mbody, 0)
    plsc.subcore_barrier()
    pltpu.sync_copy(out_sh.at[pl.ds(s * NODE_CHUNK, NODE_CHUNK), :],
                    outp_hbm.at[c, pl.ds(s * NODE_CHUNK, NODE_CHUNK), :])


def _combine_body(p_ref, b_ref, o_ref):
    o_ref[...] = p_ref[0] + p_ref[1] + b_ref[...]


_combine = pl.pallas_call(
    _combine_body,
    grid=(10,),
    in_specs=[
        pl.BlockSpec((2, 1000, C), lambda i: (0, i, 0)),
        pl.BlockSpec((1, C), lambda i: (0, 0)),
    ],
    out_specs=pl.BlockSpec((1000, C), lambda i: (i, 0)),
    out_shape=jax.ShapeDtypeStruct((N, C), jnp.float32),
)


def kernel(x, edge_index, edge_attr, W, W_edge, att_src, att_dst, att_edge, bias):
    asv = att_src.reshape(1, C)
    adv = att_dst.reshape(1, C)
    aev = att_edge.reshape(1, C)
    h, a_src2, a_dst2 = _proj(x, W, asv, adv)
    a_edge = _aedge(edge_attr.T, W_edge, aev).reshape(E)
    eflat = edge_index.reshape(2 * E)
    ex, den = _attn(a_src2.reshape(N), a_dst2.reshape(N), a_edge, eflat)
    coef = _coef(den, ex, eflat)
    outp = _agg(coef, eflat, h)
    return _combine(outp, bias.reshape(1, C))
